# baseline (speedup 1.0000x reference)
import functools

import jax
import jax.numpy as jnp
from jax import lax
from jax.experimental import pallas as pl
from jax.experimental.pallas import tpu as pltpu
from jax.experimental.pallas import tpu_sc as plsc

_NC, _NS = 2, 16
_NW = _NC * _NS
_CH = 128
_GI = 16
_NB = 4
_WCH = 128
_RPT = 640
_BN = 1000
_BNP = 1024


def _scatter_halves(h2, row3, col3, zslab):
    Dh = h2.shape[2]
    nchunk = col3.shape[1]
    rpt = _RPT
    npad = _NS * _RPT
    nwch = rpt // _WCH
    ngroup = nchunk // _GI

    mesh = plsc.VectorSubcoreMesh(core_axis_name="c", subcore_axis_name="s")

    @functools.partial(
        pl.kernel,
        out_type=jax.ShapeDtypeStruct((_NC, npad, Dh), jnp.float32),
        mesh=mesh,
        scratch_types=[
            [pltpu.VMEM((_GI, _CH), jnp.int32) for _ in range(2)],
            [pltpu.VMEM((2 * _GI, _CH // 2), jnp.int32)
             for _ in range(2)],
            [pltpu.VMEM((_CH, Dh), jnp.float32) for _ in range(_NB)],
            pltpu.VMEM_SHARED((npad, Dh), jnp.float32),
            pltpu.VMEM_SHARED((npad, Dh), jnp.float32),
            [pltpu.SemaphoreType.DMA for _ in range(_NB)],
            [pltpu.SemaphoreType.DMA for _ in range(2)],
            [pltpu.SemaphoreType.DMA for _ in range(2)],
        ],
        compiler_params=pltpu.CompilerParams(use_tc_tiling_on_sc=False),
    )
    def k(h_hbm, row_hbm, col_hbm, z_hbm, out_hbm, cbuf, rbuf,
          bufs, hsp, acc, gsem, isem, ssem):
        c = lax.axis_index("c")
        s = lax.axis_index("s")
        rbase = s * rpt

        for j in range(nwch):
            r0 = rbase + j * _WCH
            pltpu.async_copy(h_hbm.at[c, pl.ds(r0, _WCH)],
                             hsp.at[pl.ds(r0, _WCH)], gsem[0])
            pltpu.async_copy(z_hbm, acc.at[pl.ds(r0, _WCH)], gsem[1])
        for j in range(nwch):
            r0 = rbase + j * _WCH
            pltpu.make_async_copy(h_hbm.at[c, pl.ds(r0, _WCH)],
                                  hsp.at[pl.ds(r0, _WCH)], gsem[0]).wait()
            pltpu.make_async_copy(z_hbm, acc.at[pl.ds(r0, _WCH)],
                                  gsem[1]).wait()

        def load_group(gi, p):
            pltpu.async_copy(col_hbm.at[s, pl.ds(gi * _GI, _GI)], cbuf[p],
                             isem[p])
            pltpu.async_copy(row_hbm.at[s, pl.ds(gi * 2 * _GI, 2 * _GI)],
                             rbuf[p], isem[p])

        def wait_group(gi, p):
            pltpu.make_async_copy(col_hbm.at[s, pl.ds(gi * _GI, _GI)],
                                  cbuf[p], isem[p]).wait()
            pltpu.make_async_copy(row_hbm.at[s, pl.ds(gi * 2 * _GI, 2 * _GI)],
                                  rbuf[p], isem[p]).wait()

        def scatter_chunk(p, b8, b):
            h0 = bufs[b].at[pl.ds(0, _CH // 2)]
            h1 = bufs[b].at[pl.ds(_CH // 2, _CH // 2)]
            i0 = rbuf[p].at[2 * b8]
            i1 = rbuf[p].at[2 * b8 + 1]
            pltpu.async_copy(h0, acc.at[i0], ssem[0], add=True)
            pltpu.async_copy(h1, acc.at[i1], ssem[1], add=True)
            pltpu.make_async_copy(h0, acc.at[i0], ssem[0]).wait()
            pltpu.make_async_copy(h1, acc.at[i1], ssem[1]).wait()

        load_group(0, 0)
        load_group(1, 1)
        plsc.subcore_barrier()
        wait_group(0, 0)

        for b in range(_NB):
            pltpu.async_copy(hsp.at[cbuf[0].at[b]], bufs[b], gsem[b])

        def body(t, carry):
            for p in range(2):
                gi = 2 * t + p

                @pl.when(gi + 1 < ngroup)
                def _():
                    wait_group(gi + 1, 1 - p)

                for b8 in range(_GI):
                    b = b8 % _NB
                    g = gi * _GI + b8
                    pltpu.make_async_copy(hsp.at[cbuf[p].at[b8]],
                                          bufs[b], gsem[b]).wait()
                    scatter_chunk(p, b8, b)

                    nxt = b8 + _NB
                    if nxt < _GI:
                        @pl.when(g + _NB < nchunk)
                        def _():
                            pltpu.async_copy(hsp.at[cbuf[p].at[nxt]],
                                             bufs[b], gsem[b])
                    else:
                        @pl.when(g + _NB < nchunk)
                        def _():
                            pltpu.async_copy(
                                hsp.at[cbuf[1 - p].at[nxt - _GI]],
                                bufs[b], gsem[b])

                @pl.when(gi + 2 < ngroup)
                def _():
                    load_group(gi + 2, p)
            return carry

        lax.fori_loop(0, ngroup // 2, body, 0)
        plsc.subcore_barrier()

        for j in range(nwch):
            r0 = rbase + j * _WCH
            pltpu.async_copy(acc.at[pl.ds(r0, _WCH)],
                             out_hbm.at[c, pl.ds(r0, _WCH)], gsem[2])
        for j in range(nwch):
            r0 = rbase + j * _WCH
            pltpu.make_async_copy(acc.at[pl.ds(r0, _WCH)],
                                  out_hbm.at[c, pl.ds(r0, _WCH)],
                                  gsem[2]).wait()

    return k(h2, row3, col3, zslab)


def _matmul(x, W, Np):
    D = x.shape[1]
    H = W.shape[1]
    Hh = H // 2

    def kfn(x_ref, w_ref, o_ref):
        r = jnp.dot(x_ref[...], w_ref[...], preferred_element_type=jnp.float32)
        o_ref[0] = r[:, :Hh]
        o_ref[1] = r[:, Hh:]

    return pl.pallas_call(
        kfn,
        grid=(Np // _BNP,),
        in_specs=[
            pl.BlockSpec((_BNP, D), lambda i: (i, 0)),
            pl.BlockSpec((D, H), lambda i: (0, 0)),
        ],
        out_specs=pl.BlockSpec((2, _BNP, Hh), lambda i: (0, i, 0)),
        out_shape=jax.ShapeDtypeStruct((2, Np, Hh), jnp.float32),
    )(x, W)


def _mid(p, b1, g1, be1, W2):
    Np, Hh = p.shape[1], p.shape[2]
    H = 2 * Hh
    D2 = W2.shape[1]
    D2h = D2 // 2

    def kfn(p_ref, b_ref, g_ref, be_ref, w_ref, o_ref):
        s = jnp.concatenate([p_ref[0], p_ref[1]], axis=-1) + b_ref[...]
        mu = jnp.mean(s, axis=-1, keepdims=True)
        var = jnp.mean((s - mu) ** 2, axis=-1, keepdims=True)
        t = (s - mu) * lax.rsqrt(var + 1e-5) * g_ref[...] + be_ref[...]
        t = jnp.maximum(t, 0.0)
        r = jnp.dot(t, w_ref[...], preferred_element_type=jnp.float32)
        o_ref[0] = r[:, :D2h]
        o_ref[1] = r[:, D2h:]

    vec = lambda i: (0, 0)
    return pl.pallas_call(
        kfn,
        grid=(Np // _BNP,),
        in_specs=[
            pl.BlockSpec((2, _BNP, Hh), lambda i: (0, i, 0)),
            pl.BlockSpec((1, H), vec),
            pl.BlockSpec((1, H), vec),
            pl.BlockSpec((1, H), vec),
            pl.BlockSpec((H, D2), vec),
        ],
        out_specs=pl.BlockSpec((2, _BNP, D2h), lambda i: (0, i, 0)),
        out_shape=jax.ShapeDtypeStruct((2, Np, D2h), jnp.float32),
    )(p, b1.reshape(1, H), g1.reshape(1, H), be1.reshape(1, H), W2)


def _final(p, b2, g2, be2, x):
    N, D = x.shape
    Dh = p.shape[2]

    def kfn(p_ref, b_ref, g_ref, be_ref, x_ref, o_ref):
        s = jnp.concatenate([p_ref[0], p_ref[1]], axis=-1) + b_ref[...]
        mu = jnp.mean(s, axis=-1, keepdims=True)
        var = jnp.mean((s - mu) ** 2, axis=-1, keepdims=True)
        t = (s - mu) * lax.rsqrt(var + 1e-5) * g_ref[...] + be_ref[...]
        o_ref[...] = t + x_ref[...]

    vec = lambda i: (0, 0)
    return pl.pallas_call(
        kfn,
        grid=(N // _BN,),
        in_specs=[
            pl.BlockSpec((2, _BN, Dh), lambda i: (0, i, 0)),
            pl.BlockSpec((1, D), vec),
            pl.BlockSpec((1, D), vec),
            pl.BlockSpec((1, D), vec),
            pl.BlockSpec((_BN, D), lambda i: (i, 0)),
        ],
        out_specs=pl.BlockSpec((_BN, D), lambda i: (i, 0)),
        out_shape=jax.ShapeDtypeStruct((N, D), jnp.float32),
    )(p, b2.reshape(1, D), g2.reshape(1, D), be2.reshape(1, D), x)


def kernel(x, edge_index, W1, b1, g1, be1, W2, b2, g2, be2):
    N, D = x.shape
    H = W1.shape[1]
    E = edge_index.shape[1]
    zslab = jnp.zeros((_WCH, H // 2), jnp.float32)

    ept = E // _NS
    eptp = -(-ept // (2 * _GI * _CH)) * (2 * _GI * _CH)
    npad = _NS * _RPT
    row3 = jnp.full((_NS, eptp), npad - 1, jnp.int32)
    row3 = row3.at[:, :ept].set(edge_index[0].reshape(_NS, ept))
    row3 = row3.reshape(_NS, eptp // (_CH // 2), _CH // 2)
    col3 = jnp.zeros((_NS, eptp), jnp.int32)
    col3 = col3.at[:, :ept].set(edge_index[1].reshape(_NS, ept))
    col3 = col3.reshape(_NS, eptp // _CH, _CH)

    h1 = _matmul(x, W1, npad)
    p1 = _scatter_halves(h1, row3, col3, zslab)
    h2 = _mid(p1, b1, g1, be1, W2)
    p2 = _scatter_halves(h2, row3, col3, zslab)
    return _final(p2, b2, g2, be2, x)


# speedup vs baseline: 1.0012x; 1.0012x over previous
import functools

import jax
import jax.numpy as jnp
from jax import lax
from jax.experimental import pallas as pl
from jax.experimental.pallas import tpu as pltpu
from jax.experimental.pallas import tpu_sc as plsc

_NC, _NS = 2, 16
_NW = _NC * _NS
_CH = 128
_GI = 16
_NB = 4
_WCH = 128
_RPT = 640
_BN = 1000
_BNP = 1024


def _scatter_halves(h2, row3, col3, zslab):
    Dh = h2.shape[2]
    nchunk = col3.shape[1]
    rpt = _RPT
    npad = _NS * _RPT
    nwch = rpt // _WCH
    ngroup = nchunk // _GI

    mesh = plsc.VectorSubcoreMesh(core_axis_name="c", subcore_axis_name="s")

    @functools.partial(
        pl.kernel,
        out_type=jax.ShapeDtypeStruct((_NC, npad, Dh), jnp.float32),
        mesh=mesh,
        scratch_types=[
            [pltpu.VMEM((_GI, _CH), jnp.int32) for _ in range(2)],
            [pltpu.VMEM((_GI, _CH), jnp.int32) for _ in range(2)],
            [pltpu.VMEM((_CH, Dh), jnp.float32) for _ in range(_NB)],
            pltpu.VMEM_SHARED((npad, Dh), jnp.float32),
            pltpu.VMEM_SHARED((npad, Dh), jnp.float32),
            [pltpu.SemaphoreType.DMA for _ in range(_NB)],
            [pltpu.SemaphoreType.DMA for _ in range(2)],
        ],
        compiler_params=pltpu.CompilerParams(use_tc_tiling_on_sc=False),
    )
    def k(h_hbm, row_hbm, col_hbm, z_hbm, out_hbm, cbuf, rbuf,
          bufs, hsp, acc, gsem, isem):
        c = lax.axis_index("c")
        s = lax.axis_index("s")
        rbase = s * rpt

        for j in range(nwch):
            r0 = rbase + j * _WCH
            pltpu.async_copy(h_hbm.at[c, pl.ds(r0, _WCH)],
                             hsp.at[pl.ds(r0, _WCH)], gsem[0])
            pltpu.async_copy(z_hbm, acc.at[pl.ds(r0, _WCH)], gsem[1])
        for j in range(nwch):
            r0 = rbase + j * _WCH
            pltpu.make_async_copy(h_hbm.at[c, pl.ds(r0, _WCH)],
                                  hsp.at[pl.ds(r0, _WCH)], gsem[0]).wait()
            pltpu.make_async_copy(z_hbm, acc.at[pl.ds(r0, _WCH)],
                                  gsem[1]).wait()

        def load_group(gi, p):
            pltpu.async_copy(col_hbm.at[s, pl.ds(gi * _GI, _GI)], cbuf[p],
                             isem[p])
            pltpu.async_copy(row_hbm.at[s, pl.ds(gi * _GI, _GI)], rbuf[p],
                             isem[p])

        def wait_group(gi, p):
            pltpu.make_async_copy(col_hbm.at[s, pl.ds(gi * _GI, _GI)],
                                  cbuf[p], isem[p]).wait()
            pltpu.make_async_copy(row_hbm.at[s, pl.ds(gi * _GI, _GI)],
                                  rbuf[p], isem[p]).wait()

        load_group(0, 0)
        load_group(1, 1)
        plsc.subcore_barrier()
        wait_group(0, 0)

        for b in range(_NB):
            pltpu.async_copy(hsp.at[cbuf[0].at[b]], bufs[b], gsem[b])

        def body(t, carry):
            for p in range(2):
                gi = 2 * t + p

                @pl.when(gi + 1 < ngroup)
                def _():
                    wait_group(gi + 1, 1 - p)

                for b8 in range(_GI):
                    b = b8 % _NB
                    g = gi * _GI + b8
                    pltpu.make_async_copy(hsp.at[cbuf[p].at[b8]],
                                          bufs[b], gsem[b]).wait()
                    pltpu.sync_copy(bufs[b], acc.at[rbuf[p].at[b8]], add=True)

                    nxt = b8 + _NB
                    if nxt < _GI:
                        @pl.when(g + _NB < nchunk)
                        def _():
                            pltpu.async_copy(hsp.at[cbuf[p].at[nxt]],
                                             bufs[b], gsem[b])
                    else:
                        @pl.when(g + _NB < nchunk)
                        def _():
                            pltpu.async_copy(
                                hsp.at[cbuf[1 - p].at[nxt - _GI]],
                                bufs[b], gsem[b])

                @pl.when(gi + 2 < ngroup)
                def _():
                    load_group(gi + 2, p)
            return carry

        lax.fori_loop(0, ngroup // 2, body, 0)
        plsc.subcore_barrier()

        for j in range(nwch):
            r0 = rbase + j * _WCH
            pltpu.async_copy(acc.at[pl.ds(r0, _WCH)],
                             out_hbm.at[c, pl.ds(r0, _WCH)], gsem[2])
        for j in range(nwch):
            r0 = rbase + j * _WCH
            pltpu.make_async_copy(acc.at[pl.ds(r0, _WCH)],
                                  out_hbm.at[c, pl.ds(r0, _WCH)],
                                  gsem[2]).wait()

    return k(h2, row3, col3, zslab)


def _matmul(x, W, Np):
    D = x.shape[1]
    H = W.shape[1]
    Hh = H // 2

    def kfn(x_ref, w_ref, o_ref):
        r = jnp.dot(x_ref[...], w_ref[...], preferred_element_type=jnp.float32)
        o_ref[0] = r[:, :Hh]
        o_ref[1] = r[:, Hh:]

    return pl.pallas_call(
        kfn,
        grid=(Np // _BNP,),
        in_specs=[
            pl.BlockSpec((_BNP, D), lambda i: (i, 0)),
            pl.BlockSpec((D, H), lambda i: (0, 0)),
        ],
        out_specs=pl.BlockSpec((2, _BNP, Hh), lambda i: (0, i, 0)),
        out_shape=jax.ShapeDtypeStruct((2, Np, Hh), jnp.float32),
    )(x, W)


def _mid(p, b1, g1, be1, W2):
    Np, Hh = p.shape[1], p.shape[2]
    H = 2 * Hh
    D2 = W2.shape[1]
    D2h = D2 // 2

    def kfn(p_ref, b_ref, g_ref, be_ref, w_ref, o_ref):
        s = jnp.concatenate([p_ref[0], p_ref[1]], axis=-1) + b_ref[...]
        mu = jnp.mean(s, axis=-1, keepdims=True)
        var = jnp.mean((s - mu) ** 2, axis=-1, keepdims=True)
        t = (s - mu) * lax.rsqrt(var + 1e-5) * g_ref[...] + be_ref[...]
        t = jnp.maximum(t, 0.0)
        r = jnp.dot(t, w_ref[...], preferred_element_type=jnp.float32)
        o_ref[0] = r[:, :D2h]
        o_ref[1] = r[:, D2h:]

    vec = lambda i: (0, 0)
    return pl.pallas_call(
        kfn,
        grid=(Np // _BNP,),
        in_specs=[
            pl.BlockSpec((2, _BNP, Hh), lambda i: (0, i, 0)),
            pl.BlockSpec((1, H), vec),
            pl.BlockSpec((1, H), vec),
            pl.BlockSpec((1, H), vec),
            pl.BlockSpec((H, D2), vec),
        ],
        out_specs=pl.BlockSpec((2, _BNP, D2h), lambda i: (0, i, 0)),
        out_shape=jax.ShapeDtypeStruct((2, Np, D2h), jnp.float32),
    )(p, b1.reshape(1, H), g1.reshape(1, H), be1.reshape(1, H), W2)


def _final(p, b2, g2, be2, x):
    N, D = x.shape
    Dh = p.shape[2]

    def kfn(p_ref, b_ref, g_ref, be_ref, x_ref, o_ref):
        s = jnp.concatenate([p_ref[0], p_ref[1]], axis=-1) + b_ref[...]
        mu = jnp.mean(s, axis=-1, keepdims=True)
        var = jnp.mean((s - mu) ** 2, axis=-1, keepdims=True)
        t = (s - mu) * lax.rsqrt(var + 1e-5) * g_ref[...] + be_ref[...]
        o_ref[...] = t + x_ref[...]

    vec = lambda i: (0, 0)
    return pl.pallas_call(
        kfn,
        grid=(N // _BN,),
        in_specs=[
            pl.BlockSpec((2, _BN, Dh), lambda i: (0, i, 0)),
            pl.BlockSpec((1, D), vec),
            pl.BlockSpec((1, D), vec),
            pl.BlockSpec((1, D), vec),
            pl.BlockSpec((_BN, D), lambda i: (i, 0)),
        ],
        out_specs=pl.BlockSpec((_BN, D), lambda i: (i, 0)),
        out_shape=jax.ShapeDtypeStruct((N, D), jnp.float32),
    )(p, b2.reshape(1, D), g2.reshape(1, D), be2.reshape(1, D), x)


def kernel(x, edge_index, W1, b1, g1, be1, W2, b2, g2, be2):
    N, D = x.shape
    H = W1.shape[1]
    E = edge_index.shape[1]
    zslab = jnp.zeros((_WCH, H // 2), jnp.float32)

    ept = E // _NS
    eptp = -(-ept // (2 * _GI * _CH)) * (2 * _GI * _CH)
    npad = _NS * _RPT
    row3 = jnp.full((_NS, eptp), npad - 1, jnp.int32)
    row3 = row3.at[:, :ept].set(edge_index[0].reshape(_NS, ept))
    row3 = row3.reshape(_NS, eptp // _CH, _CH)
    col3 = jnp.zeros((_NS, eptp), jnp.int32)
    col3 = col3.at[:, :ept].set(edge_index[1].reshape(_NS, ept))
    col3 = col3.reshape(_NS, eptp // _CH, _CH)

    h1 = _matmul(x, W1, npad)
    p1 = _scatter_halves(h1, row3, col3, zslab)
    h2 = _mid(p1, b1, g1, be1, W2)
    p2 = _scatter_halves(h2, row3, col3, zslab)
    return _final(p2, b2, g2, be2, x)
